# baseline (speedup 1.0000x reference)
import functools

import jax
import jax.numpy as jnp
from jax import lax
from jax.experimental import pallas as pl
from jax.experimental.pallas import tpu as pltpu
from jax.experimental.pallas import tpu_sc as plsc

VOCAB = 100000
EMBD = 64
B = 1024

NC = 2
NS = 16
NW = NC * NS
BPW = B // NW

TILE_V = 2048


def _gather_body(table_hbm, idx_hbm, out_hbm, idx_v, rows_v, sem):
    wid = lax.axis_index("s") * NC + lax.axis_index("c")
    base = wid * BPW
    pltpu.sync_copy(idx_hbm.at[pl.ds(base, BPW)], idx_v)
    pltpu.async_copy(table_hbm.at[idx_v], rows_v, sem).wait()
    pltpu.sync_copy(rows_v, out_hbm.at[pl.ds(base, BPW)])


def _sc_gather(table, idx):
    mesh = plsc.VectorSubcoreMesh(core_axis_name="c", subcore_axis_name="s")
    return pl.kernel(
        _gather_body,
        mesh=mesh,
        out_type=jax.ShapeDtypeStruct((B, EMBD), jnp.float32),
        scratch_types=[
            pltpu.VMEM((BPW,), jnp.int32),
            pltpu.VMEM((BPW, EMBD), jnp.float32),
            pltpu.SemaphoreType.DMA,
        ],
        compiler_params=pltpu.CompilerParams(use_tc_tiling_on_sc=False),
    )(table, idx)


def _mm_body(emb_ref, w_ref, b_ref, out_ref):
    emb = emb_ref[...].astype(jnp.bfloat16)
    w = w_ref[...].astype(jnp.bfloat16)
    out_ref[...] = (
        jnp.dot(emb, w, preferred_element_type=jnp.float32) + b_ref[...]
    )


def _tc_matmul(tok_emb, W, b2):
    n_tiles = pl.cdiv(VOCAB, TILE_V)
    return pl.pallas_call(
        _mm_body,
        grid=(n_tiles,),
        in_specs=[
            pl.BlockSpec((B, EMBD), lambda i: (0, 0)),
            pl.BlockSpec((EMBD, TILE_V), lambda i: (0, i)),
            pl.BlockSpec((1, TILE_V), lambda i: (0, i)),
        ],
        out_specs=pl.BlockSpec((B, TILE_V), lambda i: (0, i)),
        out_shape=jax.ShapeDtypeStruct((B, VOCAB), jnp.float32),
        compiler_params=pltpu.CompilerParams(
            dimension_semantics=("arbitrary",),
        ),
    )(tok_emb, W, b2)


@jax.jit
def kernel(idx, table, W, b):
    tok_emb = jnp.take(table, idx, axis=0)
    return _tc_matmul(tok_emb, W, b.reshape(1, VOCAB))


# speedup vs baseline: 1.0011x; 1.0011x over previous
import functools

import jax
import jax.numpy as jnp
from jax import lax
from jax.experimental import pallas as pl
from jax.experimental.pallas import tpu as pltpu
from jax.experimental.pallas import tpu_sc as plsc

VOCAB = 100000
EMBD = 64
B = 1024

NC = 2
NS = 16
NW = NC * NS
BPW = B // NW

TILE_V = 2048


def _gather_body(table_hbm, idx_hbm, out_hbm, idx_v, rows_v, sem):
    wid = lax.axis_index("s") * NC + lax.axis_index("c")
    base = wid * BPW
    pltpu.sync_copy(idx_hbm.at[pl.ds(base, BPW)], idx_v)
    pltpu.async_copy(table_hbm.at[idx_v], rows_v, sem).wait()
    pltpu.sync_copy(rows_v, out_hbm.at[pl.ds(base, BPW)])


def _sc_gather(table, idx):
    mesh = plsc.VectorSubcoreMesh(core_axis_name="c", subcore_axis_name="s")
    return pl.kernel(
        _gather_body,
        mesh=mesh,
        out_type=jax.ShapeDtypeStruct((B, EMBD), jnp.float32),
        scratch_types=[
            pltpu.VMEM((BPW,), jnp.int32),
            pltpu.VMEM((BPW, EMBD), jnp.float32),
            pltpu.SemaphoreType.DMA,
        ],
        compiler_params=pltpu.CompilerParams(use_tc_tiling_on_sc=False),
    )(table, idx)


def _mm_body(emb_ref, w_ref, b_ref, out_ref):
    emb = emb_ref[...].astype(jnp.bfloat16)
    w = w_ref[...].astype(jnp.bfloat16)
    out_ref[...] = (
        jnp.dot(emb, w, preferred_element_type=jnp.float32) + b_ref[...]
    )


ROWS_TILE = 32


def _tc_matmul(tok_emb, W, b2):
    return pl.pallas_call(
        _mm_body,
        grid=(B // ROWS_TILE,),
        in_specs=[
            pl.BlockSpec((ROWS_TILE, EMBD), lambda i: (i, 0)),
            pl.BlockSpec((EMBD, VOCAB), lambda i: (0, 0)),
            pl.BlockSpec((1, VOCAB), lambda i: (0, 0)),
        ],
        out_specs=pl.BlockSpec((ROWS_TILE, VOCAB), lambda i: (i, 0)),
        out_shape=jax.ShapeDtypeStruct((B, VOCAB), jnp.float32),
        compiler_params=pltpu.CompilerParams(
            dimension_semantics=("arbitrary",),
        ),
    )(tok_emb, W, b2)


@jax.jit
def kernel(idx, table, W, b):
    tok_emb = jnp.take(table, idx, axis=0)
    return _tc_matmul(tok_emb, W, b.reshape(1, VOCAB))


# speedup vs baseline: 1.0076x; 1.0066x over previous
import functools

import jax
import jax.numpy as jnp
from jax import lax
from jax.experimental import pallas as pl
from jax.experimental.pallas import tpu as pltpu
from jax.experimental.pallas import tpu_sc as plsc

VOCAB = 100000
EMBD = 64
B = 1024

NC = 2
NS = 16
NW = NC * NS
BPW = B // NW

TILE_V = 2048


def _gather_body(table_hbm, idx_hbm, out_hbm, idx_v, rows_v, sem):
    wid = lax.axis_index("s") * NC + lax.axis_index("c")
    base = wid * BPW
    pltpu.sync_copy(idx_hbm.at[pl.ds(base, BPW)], idx_v)
    pltpu.async_copy(table_hbm.at[idx_v], rows_v, sem).wait()
    pltpu.sync_copy(rows_v, out_hbm.at[pl.ds(base, BPW)])


def _sc_gather(table, idx):
    mesh = plsc.VectorSubcoreMesh(core_axis_name="c", subcore_axis_name="s")
    return pl.kernel(
        _gather_body,
        mesh=mesh,
        out_type=jax.ShapeDtypeStruct((B, EMBD), jnp.float32),
        scratch_types=[
            pltpu.VMEM((BPW,), jnp.int32),
            pltpu.VMEM((BPW, EMBD), jnp.float32),
            pltpu.SemaphoreType.DMA,
        ],
        compiler_params=pltpu.CompilerParams(use_tc_tiling_on_sc=False),
    )(table, idx)


def _mm_body(emb_ref, w_ref, b_ref, out_ref):
    out_ref[...] = jnp.broadcast_to(b_ref[...], out_ref.shape)


ROWS_TILE = 32


def _tc_matmul(tok_emb, W, b2):
    return pl.pallas_call(
        _mm_body,
        grid=(B // ROWS_TILE,),
        in_specs=[
            pl.BlockSpec((ROWS_TILE, EMBD), lambda i: (i, 0)),
            pl.BlockSpec((EMBD, VOCAB), lambda i: (0, 0)),
            pl.BlockSpec((1, VOCAB), lambda i: (0, 0)),
        ],
        out_specs=pl.BlockSpec((ROWS_TILE, VOCAB), lambda i: (i, 0)),
        out_shape=jax.ShapeDtypeStruct((B, VOCAB), jnp.float32),
        compiler_params=pltpu.CompilerParams(
            dimension_semantics=("arbitrary",),
        ),
    )(tok_emb, W, b2)


@jax.jit
def kernel(idx, table, W, b):
    tok_emb = jnp.take(table, idx, axis=0)
    return _tc_matmul(tok_emb, W, b.reshape(1, VOCAB))


# speedup vs baseline: 1.0086x; 1.0009x over previous
import functools

import jax
import jax.numpy as jnp
from jax import lax
from jax.experimental import pallas as pl
from jax.experimental.pallas import tpu as pltpu
from jax.experimental.pallas import tpu_sc as plsc

VOCAB = 100000
EMBD = 64
B = 1024

NC = 2
NS = 16
NW = NC * NS
BPW = B // NW


def _gather_body(table_hbm, idx_hbm, out_hbm, idx_v, rows_v, sem):
    wid = lax.axis_index("s") * NC + lax.axis_index("c")
    base = wid * BPW
    pltpu.sync_copy(idx_hbm.at[pl.ds(base, BPW)], idx_v)
    pltpu.async_copy(table_hbm.at[idx_v], rows_v, sem).wait()
    pltpu.sync_copy(rows_v, out_hbm.at[pl.ds(base, BPW)])


def _sc_gather(table, idx):
    mesh = plsc.VectorSubcoreMesh(core_axis_name="c", subcore_axis_name="s")
    return pl.kernel(
        _gather_body,
        mesh=mesh,
        out_type=jax.ShapeDtypeStruct((B, EMBD), jnp.float32),
        scratch_types=[
            pltpu.VMEM((BPW,), jnp.int32),
            pltpu.VMEM((BPW, EMBD), jnp.float32),
            pltpu.SemaphoreType.DMA,
        ],
        compiler_params=pltpu.CompilerParams(use_tc_tiling_on_sc=False),
    )(table, idx)


ROWS_TILE = 16
NSTEP = B // ROWS_TILE
NBUF = 4


def _mm_body(emb_ref, w_ref, b_ref, out_hbm, scratch, sems):
    i = pl.program_id(0)
    buf = lax.rem(i, NBUF)

    @pl.when(i >= NBUF)
    def _():
        pltpu.make_async_copy(
            scratch.at[buf],
            out_hbm.at[pl.ds((i - NBUF) * ROWS_TILE, ROWS_TILE)],
            sems.at[buf],
        ).wait()

    emb = emb_ref[...].astype(jnp.bfloat16)
    w = w_ref[...].astype(jnp.bfloat16)
    scratch[buf] = (
        jnp.dot(emb, w, preferred_element_type=jnp.float32) + b_ref[...]
    )
    pltpu.make_async_copy(
        scratch.at[buf],
        out_hbm.at[pl.ds(i * ROWS_TILE, ROWS_TILE)],
        sems.at[buf],
    ).start()

    @pl.when(i == NSTEP - 1)
    def _():
        for k in range(NBUF):
            step = NSTEP - NBUF + k
            pltpu.make_async_copy(
                scratch.at[k],
                out_hbm.at[pl.ds(step * ROWS_TILE, ROWS_TILE)],
                sems.at[k],
            ).wait()


def _tc_matmul(tok_emb, W, b2):
    return pl.pallas_call(
        _mm_body,
        grid=(NSTEP,),
        in_specs=[
            pl.BlockSpec((ROWS_TILE, EMBD), lambda i: (i, 0)),
            pl.BlockSpec((EMBD, VOCAB), lambda i: (0, 0)),
            pl.BlockSpec((1, VOCAB), lambda i: (0, 0)),
        ],
        out_specs=pl.BlockSpec(memory_space=pl.ANY),
        out_shape=jax.ShapeDtypeStruct((B, VOCAB), jnp.float32),
        scratch_shapes=[
            pltpu.VMEM((NBUF, ROWS_TILE, VOCAB), jnp.float32),
            pltpu.SemaphoreType.DMA((NBUF,)),
        ],
        compiler_params=pltpu.CompilerParams(
            dimension_semantics=("arbitrary",),
        ),
    )(tok_emb, W, b2)


@jax.jit
def kernel(idx, table, W, b):
    tok_emb = jnp.take(table, idx, axis=0)
    return _tc_matmul(tok_emb, W, b.reshape(1, VOCAB))


# speedup vs baseline: 1.9637x; 1.9470x over previous
import functools

import jax
import jax.numpy as jnp
from jax import lax
from jax.experimental import pallas as pl
from jax.experimental.pallas import tpu as pltpu
from jax.experimental.pallas import tpu_sc as plsc

VOCAB = 100000
EMBD = 64
B = 1024

NC = 2
NS = 16
NW = NC * NS
BPW = B // NW


def _gather_body(table_hbm, idx_hbm, out_hbm, idx_v, rows_v, sem):
    wid = lax.axis_index("s") * NC + lax.axis_index("c")
    base = wid * BPW
    pltpu.sync_copy(idx_hbm.at[pl.ds(base, BPW)], idx_v)
    pltpu.async_copy(table_hbm.at[idx_v], rows_v, sem).wait()
    pltpu.sync_copy(rows_v, out_hbm.at[pl.ds(base, BPW)])


def _sc_gather(table, idx):
    mesh = plsc.VectorSubcoreMesh(core_axis_name="c", subcore_axis_name="s")
    return pl.kernel(
        _gather_body,
        mesh=mesh,
        out_type=jax.ShapeDtypeStruct((B, EMBD), jnp.float32),
        scratch_types=[
            pltpu.VMEM((BPW,), jnp.int32),
            pltpu.VMEM((BPW, EMBD), jnp.float32),
            pltpu.SemaphoreType.DMA,
        ],
        compiler_params=pltpu.CompilerParams(use_tc_tiling_on_sc=False),
    )(table, idx)


TILE_V = 2048


def _mm_body(emb_ref, w_ref, b_ref, out_ref):
    emb = emb_ref[...].astype(jnp.bfloat16)
    w = w_ref[...].astype(jnp.bfloat16)
    acc = lax.dot_general(
        w, emb, (((0,), (1,)), ((), ())), preferred_element_type=jnp.float32
    )
    out_ref[...] = acc + b_ref[...]


def _tc_matmul_t(tok_emb, W, bcol):
    n_tiles = pl.cdiv(VOCAB, TILE_V)
    return pl.pallas_call(
        _mm_body,
        grid=(n_tiles,),
        in_specs=[
            pl.BlockSpec((B, EMBD), lambda i: (0, 0)),
            pl.BlockSpec((EMBD, TILE_V), lambda i: (0, i)),
            pl.BlockSpec((TILE_V, 1), lambda i: (i, 0)),
        ],
        out_specs=pl.BlockSpec((TILE_V, B), lambda i: (i, 0)),
        out_shape=jax.ShapeDtypeStruct((VOCAB, B), jnp.float32),
        compiler_params=pltpu.CompilerParams(
            dimension_semantics=("arbitrary",),
        ),
    )(tok_emb, W, bcol)


@jax.jit
def kernel(idx, table, W, b):
    tok_emb = _sc_gather(table, idx.astype(jnp.int32))
    out_t = _tc_matmul_t(tok_emb, W, b.reshape(VOCAB, 1))
    return out_t.T


# speedup vs baseline: 2.5064x; 1.2763x over previous
import functools

import jax
import jax.numpy as jnp
from jax import lax
from jax.experimental import pallas as pl
from jax.experimental.pallas import tpu as pltpu
from jax.experimental.pallas import tpu_sc as plsc

VOCAB = 100000
EMBD = 64
B = 1024

NC = 2
NS = 16
NW = NC * NS
BPW = B // NW


def _gather_body(table_hbm, idx_hbm, out_hbm, idx_v, rows_v, sem):
    wid = lax.axis_index("s") * NC + lax.axis_index("c")
    base = wid * BPW
    pltpu.sync_copy(idx_hbm.at[pl.ds(base, BPW)], idx_v)
    pltpu.async_copy(table_hbm.at[idx_v], rows_v, sem).wait()
    pltpu.sync_copy(rows_v, out_hbm.at[pl.ds(base, BPW)])


def _sc_gather(table, idx):
    mesh = plsc.VectorSubcoreMesh(core_axis_name="c", subcore_axis_name="s")
    return pl.kernel(
        _gather_body,
        mesh=mesh,
        out_type=jax.ShapeDtypeStruct((B, EMBD), jnp.float32),
        scratch_types=[
            pltpu.VMEM((BPW,), jnp.int32),
            pltpu.VMEM((BPW, EMBD), jnp.float32),
            pltpu.SemaphoreType.DMA,
        ],
        compiler_params=pltpu.CompilerParams(use_tc_tiling_on_sc=False),
    )(table, idx)


TILE_V = 2048
NSTEP = pl.cdiv(VOCAB, TILE_V)
TAIL_V = VOCAB - (NSTEP - 1) * TILE_V
NBUF = 4


def _mm_body(emb_ref, w_ref, b_ref, out_hbm, scratch, sems):
    i = pl.program_id(0)
    buf = lax.rem(i, NBUF)

    @pl.when(i >= NBUF)
    def _():
        pltpu.make_async_copy(
            scratch.at[buf],
            out_hbm.at[pl.ds((i - NBUF) * TILE_V, TILE_V)],
            sems.at[buf],
        ).wait()

    emb = emb_ref[...].astype(jnp.bfloat16)
    w = w_ref[...].astype(jnp.bfloat16)
    acc = lax.dot_general(
        w, emb, (((0,), (1,)), ((), ())), preferred_element_type=jnp.float32
    )
    bcol = lax.dot_general(
        b_ref[...],
        jnp.ones((1, 1), jnp.float32),
        (((0,), (1,)), ((), ())),
        preferred_element_type=jnp.float32,
    )
    scratch[buf] = acc + bcol

    @pl.when(i < NSTEP - 1)
    def _():
        pltpu.make_async_copy(
            scratch.at[buf],
            out_hbm.at[pl.ds(i * TILE_V, TILE_V)],
            sems.at[buf],
        ).start()

    @pl.when(i == NSTEP - 1)
    def _():
        pltpu.make_async_copy(
            scratch.at[buf, pl.ds(0, TAIL_V)],
            out_hbm.at[pl.ds(i * TILE_V, TAIL_V)],
            sems.at[buf],
        ).start()
        for k in range(NBUF):
            step = NSTEP - NBUF + k
            kbuf = step % NBUF
            if step == NSTEP - 1:
                pltpu.make_async_copy(
                    scratch.at[kbuf, pl.ds(0, TAIL_V)],
                    out_hbm.at[pl.ds(step * TILE_V, TAIL_V)],
                    sems.at[kbuf],
                ).wait()
            else:
                pltpu.make_async_copy(
                    scratch.at[kbuf],
                    out_hbm.at[pl.ds(step * TILE_V, TILE_V)],
                    sems.at[kbuf],
                ).wait()


def _tc_matmul_t(tok_emb, W, brow):
    return pl.pallas_call(
        _mm_body,
        grid=(NSTEP,),
        in_specs=[
            pl.BlockSpec((B, EMBD), lambda i: (0, 0)),
            pl.BlockSpec((EMBD, TILE_V), lambda i: (0, i)),
            pl.BlockSpec((1, TILE_V), lambda i: (0, i)),
        ],
        out_specs=pl.BlockSpec(memory_space=pl.ANY),
        out_shape=jax.ShapeDtypeStruct((VOCAB, B), jnp.float32),
        scratch_shapes=[
            pltpu.VMEM((NBUF, TILE_V, B), jnp.float32),
            pltpu.SemaphoreType.DMA((NBUF,)),
        ],
        compiler_params=pltpu.CompilerParams(
            dimension_semantics=("arbitrary",),
        ),
    )(tok_emb, W, brow)


@jax.jit
def kernel(idx, table, W, b):
    tok_emb = _sc_gather(table, idx.astype(jnp.int32))
    out_t = _tc_matmul_t(tok_emb, W, b.reshape(1, VOCAB))
    return out_t.T


# speedup vs baseline: 2.5251x; 1.0075x over previous
import functools

import jax
import jax.numpy as jnp
from jax import lax
from jax.experimental import pallas as pl
from jax.experimental.pallas import tpu as pltpu
from jax.experimental.pallas import tpu_sc as plsc

VOCAB = 100000
EMBD = 64
B = 1024

NC = 2
NS = 16
NW = NC * NS
BPW = B // NW
VL = 16


def _gather_body(table2_hbm, idx_hbm, out_hbm, idx_v, idx2_v, rows_v, sem):
    wid = lax.axis_index("s") * NC + lax.axis_index("c")
    base = wid * BPW
    pltpu.sync_copy(idx_hbm.at[pl.ds(base, BPW)], idx_v)
    for j in range(BPW // VL):
        sl = pl.ds(j * VL, VL)
        idx2_v[sl] = lax.shift_right_logical(idx_v[sl], 1)
    pltpu.async_copy(table2_hbm.at[idx2_v], rows_v, sem).wait()
    pltpu.sync_copy(rows_v, out_hbm.at[pl.ds(base, BPW)])


def _sc_gather_pairs(table2, idx):
    mesh = plsc.VectorSubcoreMesh(core_axis_name="c", subcore_axis_name="s")
    return pl.kernel(
        _gather_body,
        mesh=mesh,
        out_type=jax.ShapeDtypeStruct((B, 2 * EMBD), jnp.float32),
        scratch_types=[
            pltpu.VMEM((BPW,), jnp.int32),
            pltpu.VMEM((BPW,), jnp.int32),
            pltpu.VMEM((BPW, 2 * EMBD), jnp.float32),
            pltpu.SemaphoreType.DMA,
        ],
        compiler_params=pltpu.CompilerParams(use_tc_tiling_on_sc=True),
    )(table2, idx)


TILE_V = 2048
NSTEP = pl.cdiv(VOCAB, TILE_V)
TAIL_V = VOCAB - (NSTEP - 1) * TILE_V
NBUF = 4


def _mm_body(emb2_ref, idx_ref, w_ref, b_ref, out_hbm, emb_s, scratch, sems):
    i = pl.program_id(0)
    buf = lax.rem(i, NBUF)

    @pl.when(i == 0)
    def _():
        parity = idx_ref[...] & 1
        lo = emb2_ref[:, : EMBD]
        hi = emb2_ref[:, EMBD:]
        emb_s[...] = jnp.where(parity == 0, lo, hi).astype(jnp.bfloat16)

    @pl.when(i >= NBUF)
    def _():
        pltpu.make_async_copy(
            scratch.at[buf],
            out_hbm.at[pl.ds((i - NBUF) * TILE_V, TILE_V)],
            sems.at[buf],
        ).wait()

    w = w_ref[...].astype(jnp.bfloat16)
    acc = lax.dot_general(
        w, emb_s[...], (((0,), (1,)), ((), ())),
        preferred_element_type=jnp.float32,
    )
    bcol = lax.dot_general(
        b_ref[...],
        jnp.ones((1, 1), jnp.float32),
        (((0,), (1,)), ((), ())),
        preferred_element_type=jnp.float32,
    )
    scratch[buf] = acc + bcol

    @pl.when(i < NSTEP - 1)
    def _():
        pltpu.make_async_copy(
            scratch.at[buf],
            out_hbm.at[pl.ds(i * TILE_V, TILE_V)],
            sems.at[buf],
        ).start()

    @pl.when(i == NSTEP - 1)
    def _():
        pltpu.make_async_copy(
            scratch.at[buf, pl.ds(0, TAIL_V)],
            out_hbm.at[pl.ds(i * TILE_V, TAIL_V)],
            sems.at[buf],
        ).start()
        for k in range(NBUF):
            step = NSTEP - NBUF + k
            kbuf = step % NBUF
            if step == NSTEP - 1:
                pltpu.make_async_copy(
                    scratch.at[kbuf, pl.ds(0, TAIL_V)],
                    out_hbm.at[pl.ds(step * TILE_V, TAIL_V)],
                    sems.at[kbuf],
                ).wait()
            else:
                pltpu.make_async_copy(
                    scratch.at[kbuf],
                    out_hbm.at[pl.ds(step * TILE_V, TILE_V)],
                    sems.at[kbuf],
                ).wait()


def _tc_matmul_t(emb2, idx_col, W, brow):
    return pl.pallas_call(
        _mm_body,
        grid=(NSTEP,),
        in_specs=[
            pl.BlockSpec((B, 2 * EMBD), lambda i: (0, 0)),
            pl.BlockSpec((B, 1), lambda i: (0, 0)),
            pl.BlockSpec((EMBD, TILE_V), lambda i: (0, i)),
            pl.BlockSpec((1, TILE_V), lambda i: (0, i)),
        ],
        out_specs=pl.BlockSpec(memory_space=pl.ANY),
        out_shape=jax.ShapeDtypeStruct((VOCAB, B), jnp.float32),
        scratch_shapes=[
            pltpu.VMEM((B, EMBD), jnp.bfloat16),
            pltpu.VMEM((NBUF, TILE_V, B), jnp.float32),
            pltpu.SemaphoreType.DMA((NBUF,)),
        ],
        compiler_params=pltpu.CompilerParams(
            dimension_semantics=("arbitrary",),
        ),
    )(emb2, idx_col, W, brow)


@jax.jit
def kernel(idx, table, W, b):
    idx32 = idx.astype(jnp.int32)
    table2 = table.reshape(VOCAB // 2, 2 * EMBD)
    emb2 = _sc_gather_pairs(table2, idx32)
    out_t = _tc_matmul_t(emb2, idx32.reshape(B, 1), W, b.reshape(1, VOCAB))
    return out_t.T
